# baseline (speedup 1.0000x reference)
import jax
import jax.numpy as jnp
from jax import lax
from jax.experimental import pallas as pl
from jax.experimental.pallas import tpu as pltpu
from jax.experimental.pallas import tpu_sc as plsc

B = 32
C_IN = 3
H = 512
W = 512
CH = 448
CW = 448
NUM_CROPS = 2

NW = 32
PAIRS = B * C_IN
PER_W = PAIRS // NW
KC = 32
NCHUNK = H // KC
KO = 16
NOUT = CH // KO
NI = PER_W * NCHUNK
NSLOT = 5
LANES = 16
NJ = CW // LANES


def _body(inds_hbm, in_hbm, out_hbm, inds_v, rbuf,
          ob00, ob01, ob10, ob11,
          isem, os00, os01, os10, os11):
    wid = lax.axis_index("s") * 2 + lax.axis_index("c")
    pltpu.sync_copy(inds_hbm, inds_v.at[pl.ds(0, B * NUM_CROPS * 2)])
    lanes = lax.iota(jnp.int32, LANES)
    obufs = ((ob00, ob01), (ob10, ob11))
    osems = ((os00, os01), (os10, os11))

    def plane(i):
        gp = wid * PER_W + i // NCHUNK
        return gp // C_IN, gp % C_IN, i // NCHUNK, i % NCHUNK

    def in_copy(i):
        b, c, _, m = plane(i)
        return pltpu.make_async_copy(
            in_hbm.at[b, c, pl.ds(pl.multiple_of(m * KC, 8), KC), :],
            rbuf.at[pl.ds(pl.multiple_of(i % NSLOT * KC, 8), KC), :],
            isem)

    def out_copy(i, k, n, s):
        b, c, _, _ = plane(i)
        return pltpu.make_async_copy(
            obufs[n][s],
            out_hbm.at[b * NUM_CROPS + n, c,
                       pl.ds(pl.multiple_of(k * KO, 8), KO), :],
            osems[n][s])

    def emit(i, k, n, s):
        b, _, p, _ = plane(i)
        hw = inds_v[pl.ds((b * NUM_CROPS + n) * 2, LANES)]
        h0k = hw[0] + k * KO
        colv = hw[1] + lanes
        ob = obufs[n][s]

        @plsc.parallel_loop(0, KO, unroll=8)
        def _(rr):
            h = h0k + rr
            ring_r = (p * NCHUNK + h // KC) % NSLOT * KC + h % KC
            row = jnp.full((LANES,), ring_r, jnp.int32)
            for jj in range(NJ):
                v = plsc.load_gather(rbuf, [row, colv + jj * LANES])
                ob[rr, pl.ds(jj * LANES, LANES)] = v

    in_copy(0).start()

    def grp_body(g, _):
        for s in range(2):
            i = 2 * g + s

            @pl.when(i + 1 < NI)
            def _():
                in_copy(i + 1).start()

            in_copy(i).wait()
            m = i % NCHUNK

            @pl.when(m >= 2)
            def _():
                for n in range(NUM_CROPS):
                    for dk, so in ((4, 0), (3, 1)):
                        k = 2 * m - dk
                        @pl.when(jnp.logical_or(k >= 2, i >= NCHUNK))
                        def _():
                            out_copy(i, jnp.maximum(k - 2, 0), n, so).wait()

                        emit(i, k, n, so)
                        out_copy(i, k, n, so).start()
        return 0

    lax.fori_loop(0, NI // 2, grp_body, 0)
    last = NI - 1
    for n in range(NUM_CROPS):
        out_copy(last, NOUT - 2, n, (NOUT - 2) % 2).wait()
        out_copy(last, NOUT - 1, n, (NOUT - 1) % 2).wait()


def kernel(inputs, crop_inds):
    mesh = plsc.VectorSubcoreMesh(core_axis_name="c", subcore_axis_name="s",
                                  num_cores=2, num_subcores=16)
    f = pl.kernel(
        _body,
        out_type=jax.ShapeDtypeStruct((B * NUM_CROPS, C_IN, CH, CW),
                                      jnp.float32),
        mesh=mesh,
        compiler_params=pltpu.CompilerParams(use_tc_tiling_on_sc=True,
                                             needs_layout_passes=False),
        scratch_types=(
            [pltpu.VMEM((B * NUM_CROPS * 2 + LANES,), jnp.int32),
             pltpu.VMEM((NSLOT * KC, W), jnp.float32)]
            + [pltpu.VMEM((KO, CW), jnp.float32)] * 4
            + [pltpu.SemaphoreType.DMA] * 5
        ),
    )
    return f(crop_inds.reshape(-1).astype(jnp.int32), inputs)


# speedup vs baseline: 1.2921x; 1.2921x over previous
import jax
import jax.numpy as jnp
from jax import lax
from jax.experimental import pallas as pl
from jax.experimental.pallas import tpu as pltpu
from jax.experimental.pallas import tpu_sc as plsc

B = 32
C_IN = 3
H = 512
W = 512
CH = 448
CW = 448
NUM_CROPS = 2

NW = 32
TRIPLES = B * NUM_CROPS * C_IN
PER_W = TRIPLES // NW
KR = 56
IR = KR + 8
NCHUNK = CH // KR
NU = PER_W * NCHUNK
LANES = 16
NJ = CW // LANES


def _body(inds_hbm, in_hbm, out_hbm, inds_v,
          ib0, ib1, ob0, ob1, isem0, isem1, osem0, osem1):
    wid = lax.axis_index("s") * 2 + lax.axis_index("c")
    pltpu.sync_copy(inds_hbm, inds_v.at[pl.ds(0, B * NUM_CROPS * 2)])
    lanes = lax.iota(jnp.int32, LANES)
    ibufs, obufs = (ib0, ib1), (ob0, ob1)
    isems, osems = (isem0, isem1), (osem0, osem1)

    def params(u):
        j = u // NCHUNK
        k = u % NCHUNK
        t = wid * PER_W + j
        b = t // (NUM_CROPS * C_IN)
        r = t % (NUM_CROPS * C_IN)
        n = r // C_IN
        c = r % C_IN
        hw = inds_v[pl.ds((b * NUM_CROPS + n) * 2, LANES)]
        h0 = hw[0]
        w0 = hw[1]
        h8 = pl.multiple_of((h0 // 8) * 8, 8)
        return b, n, c, k, h8, h0 - h8, w0

    def in_copy(u, s):
        b, _, c, k, h8, _, _ = params(u)
        return pltpu.make_async_copy(
            in_hbm.at[b, c, pl.ds(h8 + k * KR, IR), :],
            ibufs[s], isems[s])

    def out_copy(u, s):
        b, n, c, k, _, _, _ = params(u)
        return pltpu.make_async_copy(
            obufs[s],
            out_hbm.at[b * NUM_CROPS + n, c, pl.ds(k * KR, KR), :],
            osems[s])

    def compute(u, s):
        _, _, _, _, _, dh, w0 = params(u)
        ib, ob = ibufs[s], obufs[s]
        col0 = w0 + lanes

        @plsc.parallel_loop(0, KR, unroll=8)
        def _(rr):
            row = jnp.full((LANES,), rr + dh, jnp.int32)
            for jj in range(NJ):
                v = plsc.load_gather(ib, [row, col0 + jj * LANES])
                ob[rr, pl.ds(jj * LANES, LANES)] = v

    in_copy(0, 0).start()

    def pair_body(p, _):
        for s in range(2):
            u = 2 * p + s

            @pl.when(u + 1 < NU)
            def _():
                in_copy(u + 1, (s + 1) % 2).start()

            in_copy(u, s).wait()

            @pl.when(u >= 2)
            def _():
                out_copy(u - 2, s).wait()

            compute(u, s)
            out_copy(u, s).start()
        return 0

    lax.fori_loop(0, NU // 2, pair_body, 0)
    out_copy(NU - 2, 0).wait()
    out_copy(NU - 1, 1).wait()


def kernel(inputs, crop_inds):
    mesh = plsc.VectorSubcoreMesh(core_axis_name="c", subcore_axis_name="s",
                                  num_cores=2, num_subcores=16)
    f = pl.kernel(
        _body,
        out_type=jax.ShapeDtypeStruct((B * NUM_CROPS, C_IN, CH, CW),
                                      jnp.float32),
        mesh=mesh,
        compiler_params=pltpu.CompilerParams(use_tc_tiling_on_sc=True,
                                             needs_layout_passes=False),
        scratch_types=[
            pltpu.VMEM((B * NUM_CROPS * 2 + LANES,), jnp.int32),
            pltpu.VMEM((IR, W), jnp.float32),
            pltpu.VMEM((IR, W), jnp.float32),
            pltpu.VMEM((KR, CW), jnp.float32),
            pltpu.VMEM((KR, CW), jnp.float32),
            pltpu.SemaphoreType.DMA,
            pltpu.SemaphoreType.DMA,
            pltpu.SemaphoreType.DMA,
            pltpu.SemaphoreType.DMA,
        ],
    )
    return f(crop_inds.reshape(-1).astype(jnp.int32), inputs)
